# baseline (speedup 1.0000x reference)
import functools

import jax
import jax.numpy as jnp
from jax import lax
from jax.experimental import pallas as pl
from jax.experimental.pallas import tpu as pltpu
from jax.experimental.pallas import tpu_sc as plsc

NC = 2
NS = 16
NW = NC * NS
C = 72
NB = 2


def _make_seg_sum(N_pad, D, E):
    EPW = E // NW
    NCHUNK = EPW // C
    NOUT = NCHUNK // NB
    assert E % NW == 0 and EPW % C == 0 and NCHUNK % NB == 0
    RPT = N_pad // NS
    assert N_pad % NS == 0 and RPT % 8 == 0
    mesh = plsc.VectorSubcoreMesh(core_axis_name="c", subcore_axis_name="s")

    @functools.partial(
        pl.kernel,
        out_type=jax.ShapeDtypeStruct((NC, N_pad, D), jnp.float32),
        mesh=mesh,
        scratch_types=[
            pltpu.VMEM_SHARED((N_pad, D), jnp.float32),
            pltpu.VMEM((EPW,), jnp.int32),
            pltpu.VMEM((NCHUNK, C), jnp.int32),
            pltpu.VMEM((NB, C, D), jnp.float32),
            pltpu.SemaphoreType.DMA((NB,)),
            pltpu.SemaphoreType.DMA((NB,)),
            pltpu.SemaphoreType.DMA,
        ],
    )
    def seg(x_hbm, src_hbm, dst_hbm, zeros_hbm, out_hbm,
            acc, src_all, dst_all, rows, gsem, ssem, msem):
        cid = lax.axis_index("c")
        sid = lax.axis_index("s")
        wid = sid * NC + cid
        rbase = sid * RPT
        cz = pltpu.async_copy(zeros_hbm.at[pl.ds(rbase, RPT)],
                              acc.at[pl.ds(rbase, RPT)], msem)
        cs = pltpu.async_copy(src_hbm.at[wid], src_all, msem)
        cd = pltpu.async_copy(dst_hbm.at[wid], dst_all, msem)
        cz.wait()
        cs.wait()
        cd.wait()
        plsc.subcore_barrier()
        for b in range(NB):
            pltpu.async_copy(x_hbm.at[src_all.at[pl.ds(b * C, C)]],
                             rows.at[b], gsem.at[b])

        def body(i, carry):
            g0 = i * NB
            for b in range(NB):
                g = g0 + b
                pltpu.make_async_copy(zeros_hbm.at[pl.ds(0, C)], rows.at[b],
                                      gsem.at[b]).wait()
                pltpu.async_copy(rows.at[b], acc.at[dst_all.at[g]],
                                 ssem.at[b], add=True).wait()
                off = (g + NB) * C
                pltpu.async_copy(x_hbm.at[src_all.at[pl.ds(off, C)]],
                                 rows.at[b], gsem.at[b])
            return carry

        lax.fori_loop(0, NOUT - 1, body, 0)
        g0 = (NOUT - 1) * NB
        for b in range(NB):
            g = g0 + b
            pltpu.make_async_copy(zeros_hbm.at[pl.ds(0, C)], rows.at[b],
                                  gsem.at[b]).wait()
            pltpu.async_copy(rows.at[b], acc.at[dst_all.at[g]],
                             ssem.at[b], add=True).wait()
        plsc.subcore_barrier()
        pltpu.sync_copy(acc.at[pl.ds(rbase, RPT)],
                        out_hbm.at[cid, pl.ds(rbase, RPT)])

    return seg


def _make_mlp(N, D, H, O, final_relu):
    B = 1000
    assert N % B == 0
    grid = N // B

    def body(x_ref, agg_ref, w1_ref, b1_ref, w2_ref, b2_ref, o_ref):
        xb = x_ref[...] + agg_ref[0] + agg_ref[1]
        h = jnp.dot(xb, w1_ref[...], preferred_element_type=jnp.float32)
        h = jnp.maximum(h + b1_ref[...], 0.0)
        o = jnp.dot(h, w2_ref[...], preferred_element_type=jnp.float32)
        o = o + b2_ref[...]
        if final_relu:
            o = jnp.maximum(o, 0.0)
        o_ref[...] = o

    return pl.pallas_call(
        body,
        grid=(grid,),
        in_specs=[
            pl.BlockSpec((B, D), lambda i: (i, 0)),
            pl.BlockSpec((NC, B, D), lambda i: (0, i, 0)),
            pl.BlockSpec((D, H), lambda i: (0, 0)),
            pl.BlockSpec((1, H), lambda i: (0, 0)),
            pl.BlockSpec((H, O), lambda i: (0, 0)),
            pl.BlockSpec((1, O), lambda i: (0, 0)),
        ],
        out_specs=pl.BlockSpec((B, O), lambda i: (i, 0)),
        out_shape=jax.ShapeDtypeStruct((N, O), jnp.float32),
    )


def kernel(x, edge_index, W11, b11, W12, b12, W21, b21, W22, b22):
    N, D = x.shape
    H = W11.shape[1]
    O = W22.shape[1]
    E = edge_index.shape[1]
    N_pad = ((N + NS * 8 - 1) // (NS * 8)) * (NS * 8)
    CPW = -(-(E // NW) // (C * NB)) * (C * NB)
    E_pad = CPW * NW
    pad = E_pad - E
    src_flat = jnp.concatenate(
        [edge_index[0], jnp.zeros((pad,), jnp.int32)])
    dst_flat = jnp.concatenate(
        [edge_index[1], jnp.full((pad,), N_pad - 1, jnp.int32)])
    src = src_flat.reshape(NW, CPW)
    dst = dst_flat.reshape(NW, CPW // C, C)
    zeros = jnp.zeros((N_pad, D), jnp.float32)

    seg = _make_seg_sum(N_pad, D, E_pad)
    mlp1 = _make_mlp(N, D, H, H, final_relu=True)
    mlp2 = _make_mlp(N, H, H, O, final_relu=False)

    agg1 = seg(x, src, dst, zeros)
    h = mlp1(x, agg1, W11, b11.reshape(1, H), W12, b12.reshape(1, H))
    agg2 = seg(h, src, dst, zeros)
    out = mlp2(h, agg2, W21, b21.reshape(1, H), W22, b22.reshape(1, O))
    return out


# speedup vs baseline: 1.4441x; 1.4441x over previous
import functools

import jax
import jax.numpy as jnp
from jax import lax
from jax.experimental import pallas as pl
from jax.experimental.pallas import tpu as pltpu
from jax.experimental.pallas import tpu_sc as plsc

NC = 2
NS = 16
NW = NC * NS
C = 88
NB = 2


def _make_seg_sum(N_pad, D, E):
    EPW = E // NW
    NCHUNK = EPW // C
    NOUT = NCHUNK // NB
    assert E % NW == 0 and EPW % C == 0 and NCHUNK % NB == 0
    RPT = N_pad // NS
    assert N_pad % NS == 0 and RPT % 8 == 0
    mesh = plsc.VectorSubcoreMesh(core_axis_name="c", subcore_axis_name="s")

    @functools.partial(
        pl.kernel,
        out_type=jax.ShapeDtypeStruct((NC, N_pad, D), jnp.float32),
        mesh=mesh,
        scratch_types=[
            pltpu.VMEM_SHARED((N_pad, D), jnp.float32),
            pltpu.VMEM((EPW,), jnp.int32),
            pltpu.VMEM((NCHUNK, C), jnp.int32),
            pltpu.VMEM((NB, C, D), jnp.float32),
            pltpu.SemaphoreType.DMA((NB,)),
            pltpu.SemaphoreType.DMA((NB,)),
            pltpu.SemaphoreType.DMA,
        ],
    )
    def seg(x_hbm, src_hbm, dst_hbm, zeros_hbm, out_hbm,
            acc, src_all, dst_all, rows, gsem, ssem, msem):
        cid = lax.axis_index("c")
        sid = lax.axis_index("s")
        wid = sid * NC + cid
        rbase = sid * RPT
        cz = pltpu.async_copy(zeros_hbm.at[pl.ds(rbase, RPT)],
                              acc.at[pl.ds(rbase, RPT)], msem)
        cs = pltpu.async_copy(src_hbm.at[wid], src_all, msem)
        cd = pltpu.async_copy(dst_hbm.at[wid], dst_all, msem)
        cz.wait()
        cs.wait()
        cd.wait()
        plsc.subcore_barrier()
        for b in range(NB):
            pltpu.async_copy(x_hbm.at[src_all.at[pl.ds(b * C, C)]],
                             rows.at[b], gsem.at[b])

        def body(i, carry):
            g0 = i * NB
            for b in range(NB):
                g = g0 + b
                pltpu.make_async_copy(zeros_hbm.at[pl.ds(0, C)], rows.at[b],
                                      gsem.at[b]).wait()
                pltpu.async_copy(rows.at[b], acc.at[dst_all.at[g]],
                                 ssem.at[b], add=True).wait()
                off = (g + NB) * C
                pltpu.async_copy(x_hbm.at[src_all.at[pl.ds(off, C)]],
                                 rows.at[b], gsem.at[b])
            return carry

        lax.fori_loop(0, NOUT - 1, body, 0)
        g0 = (NOUT - 1) * NB
        for b in range(NB):
            g = g0 + b
            pltpu.make_async_copy(zeros_hbm.at[pl.ds(0, C)], rows.at[b],
                                  gsem.at[b]).wait()
            pltpu.async_copy(rows.at[b], acc.at[dst_all.at[g]],
                             ssem.at[b], add=True).wait()
        plsc.subcore_barrier()
        pltpu.sync_copy(acc.at[pl.ds(rbase, RPT)],
                        out_hbm.at[cid, pl.ds(rbase, RPT)])

    return seg


def _make_mlp(N, D, H, O, final_relu):
    B = 1000
    assert N % B == 0
    grid = N // B

    def body(x_ref, agg_ref, w1_ref, b1_ref, w2_ref, b2_ref, o_ref):
        xb = x_ref[...] + agg_ref[0] + agg_ref[1]
        h = jnp.dot(xb, w1_ref[...], preferred_element_type=jnp.float32)
        h = jnp.maximum(h + b1_ref[...], 0.0)
        o = jnp.dot(h, w2_ref[...], preferred_element_type=jnp.float32)
        o = o + b2_ref[...]
        if final_relu:
            o = jnp.maximum(o, 0.0)
        o_ref[...] = o

    return pl.pallas_call(
        body,
        grid=(grid,),
        in_specs=[
            pl.BlockSpec((B, D), lambda i: (i, 0)),
            pl.BlockSpec((NC, B, D), lambda i: (0, i, 0)),
            pl.BlockSpec((D, H), lambda i: (0, 0)),
            pl.BlockSpec((1, H), lambda i: (0, 0)),
            pl.BlockSpec((H, O), lambda i: (0, 0)),
            pl.BlockSpec((1, O), lambda i: (0, 0)),
        ],
        out_specs=pl.BlockSpec((B, O), lambda i: (i, 0)),
        out_shape=jax.ShapeDtypeStruct((N, O), jnp.float32),
    )


def kernel(x, edge_index, W11, b11, W12, b12, W21, b21, W22, b22):
    N, D = x.shape
    H = W11.shape[1]
    O = W22.shape[1]
    E = edge_index.shape[1]
    N_pad = ((N + NS * 8 - 1) // (NS * 8)) * (NS * 8)
    CPW = -(-(E // NW) // (C * NB)) * (C * NB)
    E_pad = CPW * NW
    pad = E_pad - E
    src_flat = jnp.concatenate(
        [edge_index[0], jnp.zeros((pad,), jnp.int32)])
    dst_flat = jnp.concatenate(
        [edge_index[1], jnp.full((pad,), N_pad - 1, jnp.int32)])
    src = src_flat.reshape(NW, CPW)
    dst = dst_flat.reshape(NW, CPW // C, C)
    zeros = jnp.zeros((N_pad, D), jnp.float32)

    seg = _make_seg_sum(N_pad, D, E_pad)
    mlp1 = _make_mlp(N, D, H, H, final_relu=True)
    mlp2 = _make_mlp(N, H, H, O, final_relu=False)

    agg1 = seg(x, src, dst, zeros)
    h = mlp1(x, agg1, W11, b11.reshape(1, H), W12, b12.reshape(1, H))
    agg2 = seg(h, src, dst, zeros)
    out = mlp2(h, agg2, W21, b21.reshape(1, H), W22, b22.reshape(1, O))
    return out
